# baseline (speedup 1.0000x reference)
import jax
import jax.numpy as jnp
from jax import lax
from jax.experimental import pallas as pl
from jax.experimental.pallas import tpu as pltpu
from jax.experimental.pallas import tpu_sc as plsc

VOCAB = 1024
N_ROWS = 32768
NUM_CORES = 2
NUM_SUBCORES = 16
NW = NUM_CORES * NUM_SUBCORES
ROWS_PER_TILE = N_ROWS // NW
CHUNK = 32
NCHUNK = ROWS_PER_TILE // CHUNK
LANES = 16


def _sc_onehot(ids_hbm, fill_hbm, zeros_hbm, out_hbm,
               idx_v, fill_v, buf0, buf1, sem0, sem1):
    c = lax.axis_index("c")
    s = lax.axis_index("s")
    wid = s * NUM_CORES + c
    base = wid * ROWS_PER_TILE
    pltpu.sync_copy(ids_hbm.at[pl.ds(base, ROWS_PER_TILE)], idx_v)
    pltpu.sync_copy(fill_hbm, fill_v)
    pltpu.sync_copy(zeros_hbm, buf0)
    pltpu.sync_copy(zeros_hbm, buf1)
    fill = fill_v[...]
    zero = jnp.zeros((LANES,), jnp.float32)
    rows_lo = lax.iota(jnp.int32, LANES)
    bufs = (buf0, buf1)
    sems = (sem0, sem1)
    handles = [None, None]

    def scatter_chunk(buf, j, val):
        for t in range(CHUNK // LANES):
            rows = rows_lo + t * LANES
            cols = idx_v[pl.ds(j * CHUNK + t * LANES, LANES)] % VOCAB
            flat = rows * VOCAB + cols
            plsc.store_scatter(buf, [flat], val)

    for j in range(NCHUNK):
        b = j % 2
        buf = bufs[b]
        if handles[b] is not None:
            handles[b].wait()
            scatter_chunk(buf, j - 2, zero)
        scatter_chunk(buf, j, fill)
        handles[b] = pltpu.async_copy(
            buf, out_hbm.at[pl.ds((base + j * CHUNK) * VOCAB, CHUNK * VOCAB)],
            sems[b])
    handles[0].wait()
    handles[1].wait()


def kernel(input_ids, fill_value):
    bs, seq = input_ids.shape
    ids = input_ids.reshape(N_ROWS)
    fillv = jnp.broadcast_to(fill_value.astype(jnp.float32), (LANES,))
    zeros = jnp.zeros((CHUNK * VOCAB,), jnp.float32)
    mesh = plsc.VectorSubcoreMesh(core_axis_name="c", subcore_axis_name="s")
    f = pl.kernel(
        _sc_onehot,
        out_type=jax.ShapeDtypeStruct((N_ROWS * VOCAB,), jnp.float32),
        mesh=mesh,
        compiler_params=pltpu.CompilerParams(needs_layout_passes=False),
        scratch_types=[
            pltpu.VMEM((ROWS_PER_TILE,), jnp.int32),
            pltpu.VMEM((LANES,), jnp.float32),
            pltpu.VMEM((CHUNK * VOCAB,), jnp.float32),
            pltpu.VMEM((CHUNK * VOCAB,), jnp.float32),
            pltpu.SemaphoreType.DMA,
            pltpu.SemaphoreType.DMA,
        ],
    )
    out = f(ids, fillv, zeros)
    return out.reshape(bs, seq, VOCAB)
